# baseline (speedup 1.0000x reference)
import functools
import math

import jax
import jax.numpy as jnp
from jax.experimental import pallas as pl
from jax.experimental.pallas import tpu as pltpu


def _encoder_block(idx_sm, lens_sm, emb_ref, table_ref, out_ref, rows_ref,
                   *, tb, scale):
    b = pl.program_id(0)
    t0 = pl.program_id(1) * tb

    def gather_one(t, carry):
        rows_ref[t, :] = table_ref[idx_sm[b, t0 + t], :]
        return carry

    jax.lax.fori_loop(0, tb, gather_one, 0)

    seqlen = lens_sm[b]
    tvec = t0 + jax.lax.broadcasted_iota(jnp.int32, (tb, 1), 0)
    valid = (tvec < seqlen).astype(jnp.float32)
    sin = rows_ref[...] * valid
    out_ref[...] = emb_ref[...] * scale + sin[None, :, None, :]


def kernel(embeddings, times, sequence_lengths, sinusoidal_table):
    B, T, N, E = embeddings.shape
    S = sinusoidal_table.shape[0]
    tb = 50
    scale = math.sqrt(E)

    grid_spec = pltpu.PrefetchScalarGridSpec(
        num_scalar_prefetch=2,
        grid=(B, T // tb),
        in_specs=[
            pl.BlockSpec((1, tb, N, E), lambda b, t, *_: (b, t, 0, 0)),
            pl.BlockSpec((S, E), lambda b, t, *_: (0, 0)),
        ],
        out_specs=pl.BlockSpec((1, tb, N, E), lambda b, t, *_: (b, t, 0, 0)),
        scratch_shapes=[pltpu.VMEM((tb, E), jnp.float32)],
    )

    idx = jnp.clip(jnp.round(times * 10.0).astype(jnp.int32), 0, S - 1)
    return pl.pallas_call(
        functools.partial(_encoder_block, tb=tb, scale=scale),
        grid_spec=grid_spec,
        out_shape=jax.ShapeDtypeStruct((B, T, N, E), jnp.float32),
    )(idx, sequence_lengths.astype(jnp.int32), embeddings, sinusoidal_table)


# speedup vs baseline: 1.0225x; 1.0225x over previous
import functools
import math

import jax
import jax.numpy as jnp
import numpy as np
from jax.experimental import pallas as pl
from jax.experimental.pallas import tpu as pltpu


def _encoder_block(lens_sm, emb_ref, times_ref, freq_ref, out_ref,
                   *, tb, scale, smax):
    b = pl.program_id(0)
    t0 = pl.program_id(1) * tb

    tv = times_ref[0, 0, :, :]
    idxf = jnp.clip(jnp.round(tv * 10.0), 0.0, float(smax))
    angle = idxf * freq_ref[...]
    lane = jax.lax.broadcasted_iota(jnp.int32, angle.shape, 1)
    row = jnp.where(lane % 2 == 0, jnp.sin(angle), jnp.cos(angle))

    seqlen = lens_sm[b]
    tvec = t0 + jax.lax.broadcasted_iota(jnp.int32, (tb, 1), 0)
    valid = (tvec < seqlen).astype(jnp.float32)
    sin_embed = row * valid
    out_ref[...] = emb_ref[...] * scale + sin_embed[None, :, None, :]


def kernel(embeddings, times, sequence_lengths, sinusoidal_table):
    B, T, N, E = embeddings.shape
    S = sinusoidal_table.shape[0]
    tb = 50
    scale = math.sqrt(E)

    div = np.exp(np.arange(0, E, 2, dtype=np.float32) *
                 (-math.log(10000.0) / E))
    freq = jnp.asarray(np.repeat(div, 2).reshape(1, E))

    grid_spec = pltpu.PrefetchScalarGridSpec(
        num_scalar_prefetch=1,
        grid=(B, T // tb),
        in_specs=[
            pl.BlockSpec((1, tb, N, E), lambda b, t, *_: (b, t, 0, 0)),
            pl.BlockSpec((1, 1, tb, 1), lambda b, t, *_: (b, t, 0, 0)),
            pl.BlockSpec((1, E), lambda b, t, *_: (0, 0)),
        ],
        out_specs=pl.BlockSpec((1, tb, N, E), lambda b, t, *_: (b, t, 0, 0)),
    )

    times_r = times.reshape(B, T // tb, tb, 1)
    return pl.pallas_call(
        functools.partial(_encoder_block, tb=tb, scale=scale, smax=S - 1),
        grid_spec=grid_spec,
        out_shape=jax.ShapeDtypeStruct((B, T, N, E), jnp.float32),
    )(sequence_lengths.astype(jnp.int32), embeddings, times_r, freq)


# speedup vs baseline: 1.1778x; 1.1519x over previous
import functools
import math

import jax
import jax.numpy as jnp
import numpy as np
from jax.experimental import pallas as pl
from jax.experimental.pallas import tpu as pltpu


def _encoder_block(lens_sm, emb_ref, times_ref, freq_ref, out_ref,
                   *, tb, scale, smax):
    b = pl.program_id(0)
    t0 = pl.program_id(1) * tb

    tv = times_ref[b, pl.program_id(1), :, :]
    idxf = jnp.clip(jnp.round(tv * 10.0), 0.0, float(smax))
    angle = idxf * freq_ref[...]
    lane = jax.lax.broadcasted_iota(jnp.int32, angle.shape, 1)
    row = jnp.where(lane % 2 == 0, jnp.sin(angle), jnp.cos(angle))

    seqlen = lens_sm[b]
    tvec = t0 + jax.lax.broadcasted_iota(jnp.int32, (tb, 1), 0)
    valid = (tvec < seqlen).astype(jnp.float32)
    sin_embed = row * valid
    out_ref[...] = emb_ref[...] * scale + sin_embed[None, :, None, :]


def kernel(embeddings, times, sequence_lengths, sinusoidal_table):
    B, T, N, E = embeddings.shape
    S = sinusoidal_table.shape[0]
    tb = 100
    scale = math.sqrt(E)

    div = np.exp(np.arange(0, E, 2, dtype=np.float32) *
                 (-math.log(10000.0) / E))
    freq = jnp.asarray(np.repeat(div, 2).reshape(1, E))

    grid_spec = pltpu.PrefetchScalarGridSpec(
        num_scalar_prefetch=1,
        grid=(B, T // tb),
        in_specs=[
            pl.BlockSpec((1, tb, N, E), lambda b, t, *_: (b, t, 0, 0)),
            pl.BlockSpec((B, T // tb, tb, 1), lambda b, t, *_: (0, 0, 0, 0)),
            pl.BlockSpec((1, E), lambda b, t, *_: (0, 0)),
        ],
        out_specs=pl.BlockSpec((1, tb, N, E), lambda b, t, *_: (b, t, 0, 0)),
    )

    times_r = times.reshape(B, T // tb, tb, 1)
    return pl.pallas_call(
        functools.partial(_encoder_block, tb=tb, scale=scale, smax=S - 1),
        grid_spec=grid_spec,
        out_shape=jax.ShapeDtypeStruct((B, T, N, E), jnp.float32),
    )(sequence_lengths.astype(jnp.int32), embeddings, times_r, freq)


# speedup vs baseline: 1.6396x; 1.3920x over previous
import functools
import math

import jax
import jax.numpy as jnp
import numpy as np
from jax.experimental import pallas as pl
from jax.experimental.pallas import tpu as pltpu


def _encoder_block(lens_sm, emb_ref, times_ref, freq_ref, out_ref,
                   *, n, e, scale, smax):
    b = pl.program_id(0)
    T = emb_ref.shape[1]

    tv = times_ref[b]
    idxf = jnp.clip(jnp.round(tv * 10.0), 0.0, float(smax))
    angle = idxf * freq_ref[...]
    lane = jax.lax.broadcasted_iota(jnp.int32, angle.shape, 1)
    row = jnp.where(lane % 2 == 0, jnp.sin(angle), jnp.cos(angle))

    seqlen = lens_sm[b]
    tvec = jax.lax.broadcasted_iota(jnp.int32, (T, 1), 0)
    valid = (tvec < seqlen).astype(jnp.float32)
    sin_embed = row * valid

    for i in range(n):
        sl = slice(i * e, (i + 1) * e)
        out_ref[0, :, sl] = emb_ref[0, :, sl] * scale + sin_embed


def kernel(embeddings, times, sequence_lengths, sinusoidal_table):
    B, T, N, E = embeddings.shape
    S = sinusoidal_table.shape[0]
    scale = math.sqrt(E)

    div = np.exp(np.arange(0, E, 2, dtype=np.float32) *
                 (-math.log(10000.0) / E))
    freq = jnp.asarray(np.repeat(div, 2).reshape(1, E))

    grid_spec = pltpu.PrefetchScalarGridSpec(
        num_scalar_prefetch=1,
        grid=(B,),
        in_specs=[
            pl.BlockSpec((1, T, N * E), lambda b, *_: (b, 0, 0)),
            pl.BlockSpec((B, T, 1), lambda b, *_: (0, 0, 0)),
            pl.BlockSpec((1, E), lambda b, *_: (0, 0)),
        ],
        out_specs=pl.BlockSpec((1, T, N * E), lambda b, *_: (b, 0, 0)),
    )

    out = pl.pallas_call(
        functools.partial(_encoder_block, n=N, e=E, scale=scale, smax=S - 1),
        grid_spec=grid_spec,
        out_shape=jax.ShapeDtypeStruct((B, T, N * E), jnp.float32),
    )(sequence_lengths.astype(jnp.int32), embeddings.reshape(B, T, N * E),
      times.reshape(B, T, 1), freq)
    return out.reshape(B, T, N, E)
